# baseline (speedup 1.0000x reference)
import functools

import jax
import jax.numpy as jnp
from jax import lax
from jax.experimental import pallas as pl
from jax.experimental.pallas import tpu as pltpu
from jax.experimental.pallas import tpu_sc as plsc

_D = 64
_SCALE = float(_D) ** 0.5
_NW = 32
_CHUNK = 128
_LANES = 16


def _make_kernel(n_b, n_j, n_v):
    n_bhi = n_b // _CHUNK
    nchunk = n_j * n_bhi // _NW
    mesh = plsc.VectorSubcoreMesh(core_axis_name="c", subcore_axis_name="s")

    @functools.partial(
        pl.kernel,
        mesh=mesh,
        out_type=jax.ShapeDtypeStruct((n_j, _D // 8, n_bhi, 8, _CHUNK),
                                      jnp.float32),
        scratch_types=[
            pltpu.VMEM((nchunk, _CHUNK), jnp.int32),
            pltpu.VMEM((_CHUNK, _D), jnp.float32),
            pltpu.VMEM((_CHUNK, _D), jnp.float32),
            pltpu.VMEM((_D // 8, 8, _CHUNK + 1), jnp.float32),
            pltpu.VMEM((_D // 8, 8, _CHUNK + 1), jnp.float32),
            pltpu.SemaphoreType.DMA,
            pltpu.SemaphoreType.DMA,
            pltpu.SemaphoreType.DMA,
            pltpu.SemaphoreType.DMA,
        ],
        compiler_params=pltpu.CompilerParams(use_tc_tiling_on_sc=False,
                                             needs_layout_passes=False),
    )
    def emb(idx_hbm, table_hbm, out_hbm, idx_v, rows_a, rows_b,
            stage_a, stage_b, gsem_a, gsem_b, osem_a, osem_b):
        wid = lax.axis_index("s") * 2 + lax.axis_index("c")
        base_c = wid * nchunk
        pltpu.sync_copy(idx_hbm.at[pl.ds(base_c, nchunk)], idx_v)

        iota = lax.iota(jnp.int32, _LANES)
        d_hi_vecs = [(iota + m * _LANES) // 8 for m in range(_D // _LANES)]
        d_lo_vecs = [lax.rem(iota + m * _LANES, 8) for m in range(_D // _LANES)]

        def transpose_scale(rows, stage):
            @plsc.parallel_loop(0, _CHUNK, unroll=4)
            def row_body(r):
                r_splat = jnp.broadcast_to(r, (_LANES,))
                for m in range(_D // _LANES):
                    v = rows[r, pl.ds(m * _LANES, _LANES)]
                    plsc.store_scatter(
                        stage, [d_hi_vecs[m], d_lo_vecs[m], r_splat], v)

        def start_gather(c_local, rows, sem):
            return pltpu.async_copy(
                table_hbm.at[idx_v.at[c_local]], rows, sem)

        def start_out(c_local, stage, sem):
            c = base_c + c_local
            j = c // n_bhi
            b_hi = lax.rem(c, n_bhi)
            return pltpu.async_copy(stage.at[:, :, pl.ds(0, _CHUNK)],
                                    out_hbm.at[j, :, b_hi], sem)

        def wait_gather(rows, sem):
            pltpu.make_async_copy(table_hbm.at[pl.ds(0, _CHUNK)], rows,
                                  sem).wait()

        def wait_out(stage, sem):
            pltpu.make_async_copy(stage.at[:, :, pl.ds(0, _CHUNK)],
                                  out_hbm.at[0, :, 0], sem).wait()

        start_gather(0, rows_a, gsem_a)

        def body(t, carry):
            a = 2 * t
            b = 2 * t + 1
            wait_gather(rows_a, gsem_a)
            start_gather(b, rows_b, gsem_b)

            @pl.when(t > 0)
            def _():
                wait_out(stage_a, osem_a)
            transpose_scale(rows_a, stage_a)
            start_out(a, stage_a, osem_a)

            wait_gather(rows_b, gsem_b)

            @pl.when(t < nchunk // 2 - 1)
            def _():
                start_gather(b + 1, rows_a, gsem_a)

            @pl.when(t > 0)
            def _():
                wait_out(stage_b, osem_b)
            transpose_scale(rows_b, stage_b)
            start_out(b, stage_b, osem_b)
            return carry

        lax.fori_loop(0, nchunk // 2, body, 0)
        wait_out(stage_a, osem_a)
        wait_out(stage_b, osem_b)

    return emb


def _make_relabel(n_v):
    n_vhi = n_v // 128
    n_vt = n_vhi * 128
    cvw = 128 * 31
    ncv = n_vt // cvw
    ndma = 8 * ncv
    mesh = plsc.VectorSubcoreMesh(core_axis_name="c", subcore_axis_name="s")

    @functools.partial(
        pl.kernel,
        mesh=mesh,
        out_type=jax.ShapeDtypeStruct((8 * 8, n_vt), jnp.float32),
        scratch_types=[
            pltpu.VMEM((8, 128 * 31), jnp.float32),
            pltpu.VMEM((8, 128 * 31), jnp.float32),
            pltpu.SemaphoreType.DMA,
            pltpu.SemaphoreType.DMA,
            pltpu.SemaphoreType.DMA,
            pltpu.SemaphoreType.DMA,
        ],
        compiler_params=pltpu.CompilerParams(use_tc_tiling_on_sc=True,
                                             needs_layout_passes=False),
    )
    def relabel(tt_hbm, out_hbm, buf_a, buf_b, ia, ib, oa, ob):
        wid = lax.axis_index("s") * 2 + lax.axis_index("c")

        def sl(m):
            d_hi = m // ncv
            vc = lax.rem(m, ncv)
            return (pl.ds(d_hi * 8, 8), pl.ds(vc * cvw, cvw))

        def wait_out(buf, sem):
            pltpu.make_async_copy(buf, out_hbm.at[pl.ds(0, 8),
                                                  pl.ds(0, cvw)], sem).wait()

        def body(u, carry):
            ma = wid + _NW * 2 * u
            mb = ma + _NW

            @pl.when(u > 0)
            def _():
                wait_out(buf_a, oa)
            pltpu.async_copy(tt_hbm.at[sl(ma)], buf_a, ia).wait()
            pltpu.async_copy(buf_a, out_hbm.at[sl(ma)], oa)

            @pl.when(mb < ndma)
            def _():
                @pl.when(u > 0)
                def _():
                    wait_out(buf_b, ob)
                pltpu.async_copy(tt_hbm.at[sl(mb)], buf_b, ib).wait()
                pltpu.async_copy(buf_b, out_hbm.at[sl(mb)], ob)
            return carry

        lax.fori_loop(0, (ndma // _NW + 1) // 2, body, 0)
        wait_out(buf_a, oa)
        wait_out(buf_b, ob)

    return relabel


def _make_transpose(n_v):
    n_vhi = n_v // 128
    mesh = plsc.VectorSubcoreMesh(core_axis_name="c", subcore_axis_name="s")

    @functools.partial(
        pl.kernel,
        mesh=mesh,
        out_type=jax.ShapeDtypeStruct((n_v, _D), jnp.float32),
        scratch_types=[
            pltpu.VMEM((8, 3, 8, 128), jnp.float32),
            pltpu.VMEM((8, 3, 8, 128), jnp.float32),
            pltpu.VMEM((384, _D + 1), jnp.float32),
            pltpu.VMEM((384, _D + 1), jnp.float32),
            pltpu.VMEM((_D, _D), jnp.float32),
            pltpu.SemaphoreType.DMA,
            pltpu.SemaphoreType.DMA,
            pltpu.SemaphoreType.DMA,
            pltpu.SemaphoreType.DMA,
        ],
        compiler_params=pltpu.CompilerParams(use_tc_tiling_on_sc=False,
                                             needs_layout_passes=False),
    )
    def transp(lin_hbm, tail_hbm, out_hbm, in_a, in_b, oc_a, oc_b, tail_v,
               isem_a, isem_b, osem_a, osem_b):
        wid = lax.axis_index("s") * 2 + lax.axis_index("c")
        iota = lax.iota(jnp.int32, _LANES)
        v_vecs = [iota + _LANES * k for k in range(24)]

        def start_in(c, buf, sem):
            return pltpu.async_copy(lin_hbm.at[:, pl.ds(c * 3, 3)], buf, sem)

        def wait_in(buf, sem):
            pltpu.make_async_copy(lin_hbm.at[:, pl.ds(0, 3)], buf,
                                  sem).wait()

        def start_out(c, oc, sem):
            pltpu.async_copy(oc.at[:, pl.ds(0, _D)],
                             out_hbm.at[pl.ds(c * 384, 384)], sem)

        def wait_out(oc, sem):
            pltpu.make_async_copy(oc.at[:, pl.ds(0, _D)],
                                  out_hbm.at[pl.ds(0, 384)], sem).wait()

        def transpose(buf, oc):
            @plsc.parallel_loop(0, 8, unroll=2)
            def dhi_body(d_hi):
                for cc in range(3):
                    for d_lo in range(8):
                        col_vec = jnp.broadcast_to(d_hi * 8 + d_lo,
                                                   (_LANES,))
                        for k in range(8):
                            v = buf[d_hi, cc, d_lo,
                                    pl.ds(k * _LANES, _LANES)]
                            plsc.store_scatter(
                                oc, [v_vecs[cc * 8 + k], col_vec], v * _SCALE)

        start_in(wid, in_a, isem_a)
        nch = n_vhi // 3
        niter = (nch - wid + _NW - 1) // _NW

        def body(t, carry):
            ca = wid + 2 * _NW * t
            cb = ca + _NW
            wait_in(in_a, isem_a)

            @pl.when(cb < nch)
            def _():
                start_in(cb, in_b, isem_b)

            @pl.when(t > 0)
            def _():
                wait_out(oc_a, osem_a)
            transpose(in_a, oc_a)
            start_out(ca, oc_a, osem_a)

            @pl.when(cb < nch)
            def _():
                wait_in(in_b, isem_b)

                @pl.when(ca + 2 * _NW < nch)
                def _():
                    start_in(ca + 2 * _NW, in_a, isem_a)

                @pl.when(t > 0)
                def _():
                    wait_out(oc_b, osem_b)
                transpose(in_b, oc_b)
                start_out(cb, oc_b, osem_b)
            return carry

        lax.fori_loop(0, (niter + 1) // 2, body, 0)
        wait_out(oc_a, osem_a)

        @pl.when(niter > 1)
        def _():
            wait_out(oc_b, osem_b)

        @pl.when(wid == 0)
        def _():
            pltpu.sync_copy(tail_hbm, tail_v)

            @plsc.parallel_loop(0, _D, unroll=4)
            def tail_body(u):
                for m in range(_D // _LANES):
                    oc_a[u, pl.ds(m * _LANES, _LANES)] = tail_v[
                        u, pl.ds(m * _LANES, _LANES)] * _SCALE
            pltpu.sync_copy(oc_a.at[pl.ds(0, _D), pl.ds(0, _D)],
                            out_hbm.at[pl.ds(n_vhi * 128, _D)])

    return transp


@jax.jit
def kernel(x, table):
    n_b, n_j = x.shape
    n_v = table.shape[0]
    idx2 = x.T.reshape(n_j * (n_b // _CHUNK), _CHUNK)
    n_vhi = n_v // 128
    lin2 = _make_relabel(n_v)(table.T)
    lin4 = lin2.reshape(8, 8, n_vhi, 128).transpose(0, 2, 1, 3)
    tail = table[n_v - n_v % 128:, :]
    t_lin = _make_transpose(n_v)(lin4, tail)
    out5 = _make_kernel(n_b, n_j, n_v)(idx2, t_lin)
    out = out5.transpose(2, 4, 0, 1, 3).reshape(n_b, n_j, _D)
    return out


# speedup vs baseline: 1.6544x; 1.6544x over previous
import functools

import jax
import jax.numpy as jnp
from jax import lax
from jax.experimental import pallas as pl
from jax.experimental.pallas import tpu as pltpu
from jax.experimental.pallas import tpu_sc as plsc

_D = 64
_SCALE = float(_D) ** 0.5
_NW = 32
_CHUNK = 128
_LANES = 16


def _make_kernel(n_b, n_j, n_v):
    n_bhi = n_b // _CHUNK
    nchunk = n_j * n_bhi // _NW
    mesh = plsc.VectorSubcoreMesh(core_axis_name="c", subcore_axis_name="s")

    @functools.partial(
        pl.kernel,
        mesh=mesh,
        out_type=jax.ShapeDtypeStruct((n_j, _D // 8, n_bhi, 8, _CHUNK),
                                      jnp.float32),
        scratch_types=[
            pltpu.VMEM((nchunk, _CHUNK), jnp.int32),
            pltpu.VMEM((_CHUNK, _D), jnp.float32),
            pltpu.VMEM((_CHUNK, _D), jnp.float32),
            pltpu.VMEM((_D // 8, 8, _CHUNK + 1), jnp.float32),
            pltpu.VMEM((_D // 8, 8, _CHUNK + 1), jnp.float32),
            pltpu.SemaphoreType.DMA,
            pltpu.SemaphoreType.DMA,
            pltpu.SemaphoreType.DMA,
            pltpu.SemaphoreType.DMA,
        ],
        compiler_params=pltpu.CompilerParams(use_tc_tiling_on_sc=False,
                                             needs_layout_passes=False),
    )
    def emb(idx_hbm, table_hbm, out_hbm, idx_v, rows_a, rows_b,
            stage_a, stage_b, gsem_a, gsem_b, osem_a, osem_b):
        wid = lax.axis_index("s") * 2 + lax.axis_index("c")
        base_c = wid * nchunk
        pltpu.sync_copy(idx_hbm.at[pl.ds(base_c, nchunk)], idx_v)

        iota = lax.iota(jnp.int32, _LANES)
        d_hi_vecs = [(iota + m * _LANES) // 8 for m in range(_D // _LANES)]
        d_lo_vecs = [lax.rem(iota + m * _LANES, 8) for m in range(_D // _LANES)]

        def transpose_scale(rows, stage):
            @plsc.parallel_loop(0, _CHUNK, unroll=4)
            def row_body(r):
                r_splat = jnp.broadcast_to(r, (_LANES,))
                for m in range(_D // _LANES):
                    v = rows[r, pl.ds(m * _LANES, _LANES)] * _SCALE
                    plsc.store_scatter(
                        stage, [d_hi_vecs[m], d_lo_vecs[m], r_splat], v)

        def start_gather(c_local, rows, sem):
            return pltpu.async_copy(
                table_hbm.at[idx_v.at[c_local]], rows, sem)

        def start_out(c_local, stage, sem):
            c = base_c + c_local
            j = c // n_bhi
            b_hi = lax.rem(c, n_bhi)
            return pltpu.async_copy(stage.at[:, :, pl.ds(0, _CHUNK)],
                                    out_hbm.at[j, :, b_hi], sem)

        def wait_gather(rows, sem):
            pltpu.make_async_copy(table_hbm.at[pl.ds(0, _CHUNK)], rows,
                                  sem).wait()

        def wait_out(stage, sem):
            pltpu.make_async_copy(stage.at[:, :, pl.ds(0, _CHUNK)],
                                  out_hbm.at[0, :, 0], sem).wait()

        start_gather(0, rows_a, gsem_a)

        def body(t, carry):
            a = 2 * t
            b = 2 * t + 1
            wait_gather(rows_a, gsem_a)
            start_gather(b, rows_b, gsem_b)

            @pl.when(t > 0)
            def _():
                wait_out(stage_a, osem_a)
            transpose_scale(rows_a, stage_a)
            start_out(a, stage_a, osem_a)

            wait_gather(rows_b, gsem_b)

            @pl.when(t < nchunk // 2 - 1)
            def _():
                start_gather(b + 1, rows_a, gsem_a)

            @pl.when(t > 0)
            def _():
                wait_out(stage_b, osem_b)
            transpose_scale(rows_b, stage_b)
            start_out(b, stage_b, osem_b)
            return carry

        lax.fori_loop(0, nchunk // 2, body, 0)
        wait_out(stage_a, osem_a)
        wait_out(stage_b, osem_b)

    return emb


@jax.jit
def kernel(x, table):
    n_b, n_j = x.shape
    n_v = table.shape[0]
    idx2 = x.T.reshape(n_j * (n_b // _CHUNK), _CHUNK)
    out5 = _make_kernel(n_b, n_j, n_v)(idx2, table)
    out = out5.transpose(2, 4, 0, 1, 3).reshape(n_b, n_j, _D)
    return out
